# baseline (speedup 1.0000x reference)
import functools

import jax
import jax.numpy as jnp
from jax import lax
from jax.experimental import pallas as pl
from jax.experimental.pallas import tpu as pltpu
from jax.experimental.pallas import tpu_sc as plsc

NC = 2
NS = 16
LN = 16
KE = 128
ZB = 158


def _n_pad(n):
    return ((n + NS * 8 - 1) // (NS * 8)) * (NS * 8)


def _fill_loop(ref, rows, cols, value):
    def body(i, _):
        for j in range(cols // LN):
            ref[i, pl.ds(j * LN, LN)] = jnp.full((LN,), value, jnp.float32)
        return 0
    lax.fori_loop(0, rows, body, 0)


def _make_sc_agg(n_nodes, nchunk, dc, groups, with_count=False):
    npad = _n_pad(n_nodes)
    zrows = npad // NS
    opt = npad // NS
    mesh = plsc.VectorSubcoreMesh(core_axis_name="c", subcore_axis_name="s",
                                  num_cores=NC, num_subcores=NS)

    out_type = [jax.ShapeDtypeStruct((2 * groups * npad, dc), jnp.float32)]
    scratch = [
        pltpu.VMEM((nchunk, KE), jnp.int32),
        pltpu.VMEM((nchunk, KE), jnp.int32),
        pltpu.VMEM((KE, dc), jnp.float32),
        pltpu.VMEM((KE, dc), jnp.float32),
        pltpu.VMEM((ZB, dc), jnp.float32),
        pltpu.VMEM_SHARED((npad, dc), jnp.float32),
        pltpu.SemaphoreType.DMA,
        pltpu.SemaphoreType.DMA,
    ]
    if with_count:
        assert groups == 1
        out_type.append(jax.ShapeDtypeStruct((2 * npad, LN), jnp.float32))
        scratch += [
            pltpu.VMEM((KE, LN), jnp.float32),
            pltpu.VMEM((ZB, LN), jnp.float32),
            pltpu.VMEM_SHARED((npad, LN), jnp.float32),
        ]

    @functools.partial(
        pl.kernel, mesh=mesh,
        out_type=tuple(out_type) if with_count else out_type[0],
        compiler_params=pltpu.CompilerParams(use_tc_tiling_on_sc=False),
        scratch_types=scratch,
    )
    def agg(src_hbm, dst_hbm, h_hbm, *rest):
        if with_count:
            (out_hbm, cnt_hbm, srcb, dstb, rows0, rows1, zbuf, acc, sem0,
             sem1, ones, zbuf2, acc_cnt) = rest
        else:
            out_hbm, srcb, dstb, rows0, rows1, zbuf, acc, sem0, sem1 = rest
        c = lax.axis_index("c")
        s = lax.axis_index("s")
        npairs = nchunk // 2
        _fill_loop(zbuf, ZB, dc, 0.0)
        if with_count:
            _fill_loop(ones, KE, LN, 1.0)
            _fill_loop(zbuf2, ZB, LN, 0.0)
            for t in range(zrows // ZB):
                pltpu.sync_copy(zbuf2, acc_cnt.at[pl.ds(s * zrows + t * ZB, ZB)])
        pltpu.sync_copy(dst_hbm.at[s], dstb)
        for g in range(groups):
            q = 2 * g + c
            for t in range(zrows // ZB):
                pltpu.sync_copy(zbuf, acc.at[pl.ds(s * zrows + t * ZB, ZB)])
            pltpu.sync_copy(src_hbm.at[q, s], srcb)
            plsc.subcore_barrier()

            pltpu.async_copy(h_hbm.at[srcb.at[0]], rows0, sem0)

            def chunk(j, _):
                i0 = 2 * j
                i1 = 2 * j + 1
                pltpu.make_async_copy(h_hbm.at[srcb.at[i0]], rows0, sem0).wait()
                pltpu.async_copy(h_hbm.at[srcb.at[i1]], rows1, sem1)
                pltpu.sync_copy(rows0, acc.at[dstb.at[i0]], add=True)
                if with_count:
                    pltpu.sync_copy(ones, acc_cnt.at[dstb.at[i0]], add=True)
                pltpu.make_async_copy(h_hbm.at[srcb.at[i1]], rows1, sem1).wait()

                @pl.when(j + 1 < npairs)
                def _():
                    pltpu.async_copy(h_hbm.at[srcb.at[i0 + 2]], rows0, sem0)
                pltpu.sync_copy(rows1, acc.at[dstb.at[i1]], add=True)
                if with_count:
                    pltpu.sync_copy(ones, acc_cnt.at[dstb.at[i1]], add=True)
                return 0
            lax.fori_loop(0, npairs, chunk, 0)
            plsc.subcore_barrier()
            pltpu.sync_copy(acc.at[pl.ds(s * opt, opt)],
                            out_hbm.at[pl.ds(q * npad + s * opt, opt)])
            if with_count:
                pltpu.sync_copy(acc_cnt.at[pl.ds(s * opt, opt)],
                                cnt_hbm.at[pl.ds(c * npad + s * opt, opt)])
            plsc.subcore_barrier()

    return agg


def _dot(a, b):
    return jnp.dot(a, b, preferred_element_type=jnp.float32)


def _tc_layer(x, aggq, deg, ws, wn, b, *, relu, split_h, wn_next=None,
              split_next=0):
    n, d_in = x.shape
    d_out = ws.shape[1]
    q_in, npad, dcq = aggq.shape
    rt = 1000
    grid = (n // rt,)
    outs = [jax.ShapeDtypeStruct((n, d_out), jnp.float32)]
    if split_h:
        outs.append(jax.ShapeDtypeStruct((split_h, n, d_out // split_h),
                                         jnp.float32))
    if wn_next is not None:
        d_nx = wn_next.shape[1]
        outs.append(jax.ShapeDtypeStruct((split_next, n, d_nx // split_next),
                                         jnp.float32))

    def body(x_ref, agg_ref, deg_ref, ws_ref, wn_ref, b_ref, wnx_ref, h_ref,
             *rest):
        hn = jnp.concatenate([agg_ref[q] for q in range(q_in)], axis=1)
        invd = 1.0 / jnp.maximum(deg_ref[...], 1.0)
        h = _dot(x_ref[...], ws_ref[...]) + _dot(hn * invd, wn_ref[...])
        h = h + b_ref[...]
        if relu:
            h = jnp.maximum(h, 0.0)
        h_ref[...] = h
        rest = list(rest)
        if split_h:
            hs_ref = rest.pop(0)
            w = d_out // split_h
            for q in range(split_h):
                hs_ref[q] = h[:, q * w:(q + 1) * w]
        if wn_next is not None:
            ps_ref = rest.pop(0)
            p = _dot(h, wnx_ref[...])
            w = p.shape[1] // split_next
            for q in range(split_next):
                ps_ref[q] = p[:, q * w:(q + 1) * w]

    wnx = wn_next if wn_next is not None else jnp.zeros((d_out, 2), jnp.float32)
    out_specs = [pl.BlockSpec((rt, d_out), lambda i: (i, 0))]
    if split_h:
        out_specs.append(pl.BlockSpec((split_h, rt, d_out // split_h),
                                      lambda i: (0, i, 0)))
    if wn_next is not None:
        out_specs.append(pl.BlockSpec((split_next, rt,
                                       wn_next.shape[1] // split_next),
                                      lambda i: (0, i, 0)))
    return pl.pallas_call(
        body,
        grid=grid,
        in_specs=[
            pl.BlockSpec((rt, d_in), lambda i: (i, 0)),
            pl.BlockSpec((q_in, rt, dcq), lambda i: (0, i, 0)),
            pl.BlockSpec((rt, 1), lambda i: (i, 0)),
            pl.BlockSpec(ws.shape, lambda i: (0, 0)),
            pl.BlockSpec(wn.shape, lambda i: (0, 0)),
            pl.BlockSpec((1, d_out), lambda i: (0, 0)),
            pl.BlockSpec(wnx.shape, lambda i: (0, 0)),
        ],
        out_specs=out_specs,
        out_shape=outs,
    )(x, aggq, deg, ws, wn, b.reshape(1, -1), wnx)


def _tc_layer2(h1, aggq, deg, ws, b):
    n, d_in = h1.shape
    d_out = ws.shape[1]
    q_in, npad, dcq = aggq.shape
    rt = 1000
    grid = (n // rt,)

    def body(h_ref, agg_ref, deg_ref, ws_ref, b_ref, o_ref):
        hn = jnp.concatenate([agg_ref[q] for q in range(q_in)], axis=1)
        invd = 1.0 / jnp.maximum(deg_ref[...], 1.0)
        o_ref[...] = _dot(h_ref[...], ws_ref[...]) + hn * invd + b_ref[...]

    return pl.pallas_call(
        body,
        grid=grid,
        in_specs=[
            pl.BlockSpec((rt, d_in), lambda i: (i, 0)),
            pl.BlockSpec((q_in, rt, dcq), lambda i: (0, i, 0)),
            pl.BlockSpec((rt, 1), lambda i: (i, 0)),
            pl.BlockSpec(ws.shape, lambda i: (0, 0)),
            pl.BlockSpec((1, d_out), lambda i: (0, 0)),
        ],
        out_specs=pl.BlockSpec((rt, d_out), lambda i: (i, 0)),
        out_shape=jax.ShapeDtypeStruct((n, d_out), jnp.float32),
    )(h1, aggq, deg, ws, b.reshape(1, -1))


def _split_stack(h, parts):
    n, d = h.shape
    w = d // parts
    return jnp.transpose(h.reshape(n, parts, w), (1, 0, 2)).reshape(parts * n, w)


def kernel(inputs, edge_index, W_self0, W_neigh0, b0, W_self1, W_neigh1, b1,
           W_self2, W_neigh2, b2):
    x = inputs
    n, d_in = x.shape
    e = edge_index.shape[1]
    e_pad = ((e + 2 * NS * KE - 1) // (2 * NS * KE)) * (2 * NS * KE)
    nchunk = e_pad // (NS * KE)
    npad = _n_pad(n)

    src = edge_index[0]
    dst = edge_index[1]
    pad = e_pad - e
    srcp = jnp.concatenate([src, jnp.zeros((pad,), jnp.int32)])
    dstp = jnp.concatenate([dst, jnp.full((pad,), n, jnp.int32)])
    dst3 = dstp.reshape(NS, nchunk, KE)

    offs = jnp.arange(4, dtype=jnp.int32)[:, None] * n
    src_q4 = (srcp[None, :] + offs).reshape(4, NS, nchunk, KE)

    agg0, cnt = _make_sc_agg(n, nchunk, d_in // 2, 1, with_count=True)(
        src_q4, dst3, _split_stack(x, 2))
    deg = cnt[:n, 0:1]
    h0, h0q = _tc_layer(x, agg0.reshape(2, npad, d_in // 2), deg,
                        W_self0, W_neigh0, b0, relu=True, split_h=4)

    dh = h0.shape[1]
    agg1 = _make_sc_agg(n, nchunk, dh // 4, 2)(src_q4, dst3,
                                               h0q.reshape(4 * n, dh // 4))
    h1, p2q = _tc_layer(h0, agg1.reshape(4, npad, dh // 4), deg,
                        W_self1, W_neigh1, b1, relu=True, split_h=0,
                        wn_next=W_neigh2, split_next=2)

    d2 = W_neigh2.shape[1]
    agg2 = _make_sc_agg(n, nchunk, d2 // 2, 1)(src_q4, dst3,
                                               p2q.reshape(2 * n, d2 // 2))
    h2 = _tc_layer2(h1, agg2.reshape(2, npad, d2 // 2), deg, W_self2, b2)

    return (h2, h0, h1)


# speedup vs baseline: 1.0204x; 1.0204x over previous
import functools

import jax
import jax.numpy as jnp
from jax import lax
from jax.experimental import pallas as pl
from jax.experimental.pallas import tpu as pltpu
from jax.experimental.pallas import tpu_sc as plsc

NC = 2
NS = 16
LN = 16
KE = 128
ZB = 158


def _n_pad(n):
    return ((n + NS * 8 - 1) // (NS * 8)) * (NS * 8)


def _fill_loop(ref, rows, cols, value):
    def body(i, _):
        for j in range(cols // LN):
            ref[i, pl.ds(j * LN, LN)] = jnp.full((LN,), value, jnp.float32)
        return 0
    lax.fori_loop(0, rows, body, 0)


def _make_sc_agg(n_nodes, nchunk, dc, groups, with_count=False):
    npad = _n_pad(n_nodes)
    zrows = npad // NS
    opt = npad // NS
    mesh = plsc.VectorSubcoreMesh(core_axis_name="c", subcore_axis_name="s",
                                  num_cores=NC, num_subcores=NS)

    out_type = [jax.ShapeDtypeStruct((2 * groups * npad, dc), jnp.float32)]
    scratch = [
        pltpu.VMEM((nchunk, KE), jnp.int32),
        pltpu.VMEM((nchunk, KE), jnp.int32),
        pltpu.VMEM((KE, dc), jnp.float32),
        pltpu.VMEM((KE, dc), jnp.float32),
        pltpu.VMEM((ZB, dc), jnp.float32),
        pltpu.VMEM_SHARED((npad, dc), jnp.float32),
        pltpu.SemaphoreType.DMA,
        pltpu.SemaphoreType.DMA,
    ]
    if with_count:
        assert groups == 1
        out_type.append(jax.ShapeDtypeStruct((2 * npad, LN), jnp.float32))
        scratch += [
            pltpu.VMEM((KE, LN), jnp.float32),
            pltpu.VMEM((ZB, LN), jnp.float32),
            pltpu.VMEM_SHARED((npad, LN), jnp.float32),
        ]

    @functools.partial(
        pl.kernel, mesh=mesh,
        out_type=tuple(out_type) if with_count else out_type[0],
        compiler_params=pltpu.CompilerParams(use_tc_tiling_on_sc=False),
        scratch_types=scratch,
    )
    def agg(src_hbm, dst_hbm, h_hbm, *rest):
        if with_count:
            (out_hbm, cnt_hbm, srcb, dstb, rows0, rows1, zbuf, acc, sem0,
             sem1, ones, zbuf2, acc_cnt) = rest
        else:
            out_hbm, srcb, dstb, rows0, rows1, zbuf, acc, sem0, sem1 = rest
        c = lax.axis_index("c")
        s = lax.axis_index("s")
        npairs = nchunk // 2
        _fill_loop(zbuf, ZB, dc, 0.0)
        if with_count:
            _fill_loop(ones, KE, LN, 1.0)
            _fill_loop(zbuf2, ZB, LN, 0.0)
            for t in range(zrows // ZB):
                pltpu.sync_copy(zbuf2, acc_cnt.at[pl.ds(s * zrows + t * ZB, ZB)])
        pltpu.sync_copy(dst_hbm.at[s], dstb)
        for g in range(groups):
            q = 2 * g + c
            for t in range(zrows // ZB):
                pltpu.sync_copy(zbuf, acc.at[pl.ds(s * zrows + t * ZB, ZB)])
            pltpu.sync_copy(src_hbm.at[q, s], srcb)
            plsc.subcore_barrier()

            pltpu.async_copy(h_hbm.at[srcb.at[0]], rows0, sem0)

            def chunk(j, _):
                i0 = 2 * j
                i1 = 2 * j + 1
                pltpu.make_async_copy(h_hbm.at[srcb.at[i0]], rows0, sem0).wait()
                pltpu.async_copy(h_hbm.at[srcb.at[i1]], rows1, sem1)
                pltpu.sync_copy(rows0, acc.at[dstb.at[i0]], add=True)
                if with_count:
                    pltpu.sync_copy(ones, acc_cnt.at[dstb.at[i0]], add=True)
                pltpu.make_async_copy(h_hbm.at[srcb.at[i1]], rows1, sem1).wait()

                @pl.when(j + 1 < npairs)
                def _():
                    pltpu.async_copy(h_hbm.at[srcb.at[i0 + 2]], rows0, sem0)
                pltpu.sync_copy(rows1, acc.at[dstb.at[i1]], add=True)
                if with_count:
                    pltpu.sync_copy(ones, acc_cnt.at[dstb.at[i1]], add=True)
                return 0
            lax.fori_loop(0, npairs, chunk, 0)
            plsc.subcore_barrier()
            pltpu.sync_copy(acc.at[pl.ds(s * opt, opt)],
                            out_hbm.at[pl.ds(q * npad + s * opt, opt)])
            if with_count:
                pltpu.sync_copy(acc_cnt.at[pl.ds(s * opt, opt)],
                                cnt_hbm.at[pl.ds(c * npad + s * opt, opt)])
            plsc.subcore_barrier()

    return agg


def _dot(a, b):
    return jnp.dot(a, b, preferred_element_type=jnp.float32)


def _tc_self(x, ws, b):
    n, d_in = x.shape
    d_out = ws.shape[1]
    rt = 1000
    grid = (n // rt,)

    def body(x_ref, ws_ref, b_ref, o_ref):
        o_ref[...] = _dot(x_ref[...], ws_ref[...]) + b_ref[...]

    return pl.pallas_call(
        body,
        grid=grid,
        in_specs=[
            pl.BlockSpec((rt, d_in), lambda i: (i, 0)),
            pl.BlockSpec(ws.shape, lambda i: (0, 0)),
            pl.BlockSpec((1, d_out), lambda i: (0, 0)),
        ],
        out_specs=pl.BlockSpec((rt, d_out), lambda i: (i, 0)),
        out_shape=jax.ShapeDtypeStruct((n, d_out), jnp.float32),
    )(x, ws, b.reshape(1, -1))


def _tc_combine(s, aggq, deg, wn, *, relu, split_h, wn_next=None,
                split_next=0, add_direct=False):
    n, d_out = s.shape
    q_in, npad, dcq = aggq.shape
    rt = 1000
    grid = (n // rt,)
    outs = [jax.ShapeDtypeStruct((n, d_out), jnp.float32)]
    if split_h:
        outs.append(jax.ShapeDtypeStruct((split_h, n, d_out // split_h),
                                         jnp.float32))
    if wn_next is not None:
        outs.append(jax.ShapeDtypeStruct((split_next, n,
                                          wn_next.shape[1] // split_next),
                                         jnp.float32))

    def body(s_ref, agg_ref, deg_ref, wn_ref, wnx_ref, h_ref, *rest):
        hn = jnp.concatenate([agg_ref[q] for q in range(q_in)], axis=1)
        invd = 1.0 / jnp.maximum(deg_ref[...], 1.0)
        if add_direct:
            h = s_ref[...] + hn * invd
        else:
            h = s_ref[...] + _dot(hn * invd, wn_ref[...])
        if relu:
            h = jnp.maximum(h, 0.0)
        h_ref[...] = h
        rest = list(rest)
        if split_h:
            hs_ref = rest.pop(0)
            w = d_out // split_h
            for q in range(split_h):
                hs_ref[q] = h[:, q * w:(q + 1) * w]
        if wn_next is not None:
            ps_ref = rest.pop(0)
            p = _dot(h, wnx_ref[...])
            w = p.shape[1] // split_next
            for q in range(split_next):
                ps_ref[q] = p[:, q * w:(q + 1) * w]

    wn_ = wn if wn is not None else jnp.zeros((2, d_out), jnp.float32)
    wnx = wn_next if wn_next is not None else jnp.zeros((d_out, 2), jnp.float32)
    out_specs = [pl.BlockSpec((rt, d_out), lambda i: (i, 0))]
    if split_h:
        out_specs.append(pl.BlockSpec((split_h, rt, d_out // split_h),
                                      lambda i: (0, i, 0)))
    if wn_next is not None:
        out_specs.append(pl.BlockSpec((split_next, rt,
                                       wn_next.shape[1] // split_next),
                                      lambda i: (0, i, 0)))
    return pl.pallas_call(
        body,
        grid=grid,
        in_specs=[
            pl.BlockSpec((rt, d_out), lambda i: (i, 0)),
            pl.BlockSpec((q_in, rt, dcq), lambda i: (0, i, 0)),
            pl.BlockSpec((rt, 1), lambda i: (i, 0)),
            pl.BlockSpec(wn_.shape, lambda i: (0, 0)),
            pl.BlockSpec(wnx.shape, lambda i: (0, 0)),
        ],
        out_specs=out_specs,
        out_shape=outs,
    )(s, aggq, deg, wn_, wnx)


def _split_stack(h, parts):
    n, d = h.shape
    w = d // parts
    return jnp.transpose(h.reshape(n, parts, w), (1, 0, 2)).reshape(parts * n, w)


def kernel(inputs, edge_index, W_self0, W_neigh0, b0, W_self1, W_neigh1, b1,
           W_self2, W_neigh2, b2):
    x = inputs
    n, d_in = x.shape
    e = edge_index.shape[1]
    e_pad = ((e + 2 * NS * KE - 1) // (2 * NS * KE)) * (2 * NS * KE)
    nchunk = e_pad // (NS * KE)
    npad = _n_pad(n)

    src = edge_index[0]
    dst = edge_index[1]
    pad = e_pad - e
    srcp = jnp.concatenate([src, jnp.zeros((pad,), jnp.int32)])
    dstp = jnp.concatenate([dst, jnp.full((pad,), n, jnp.int32)])
    dst3 = dstp.reshape(NS, nchunk, KE)

    offs = jnp.arange(4, dtype=jnp.int32)[:, None] * n
    src_q4 = (srcp[None, :] + offs).reshape(4, NS, nchunk, KE)

    s0 = _tc_self(x, W_self0, b0)

    agg0, cnt = _make_sc_agg(n, nchunk, d_in // 2, 1, with_count=True)(
        src_q4, dst3, _split_stack(x, 2))
    deg = cnt[:n, 0:1]
    h0, h0q = _tc_combine(s0, agg0.reshape(2, npad, d_in // 2), deg,
                          W_neigh0, relu=True, split_h=4)

    dh = h0.shape[1]
    s1 = _tc_self(h0, W_self1, b1)
    agg1 = _make_sc_agg(n, nchunk, dh // 4, 2)(src_q4, dst3,
                                               h0q.reshape(4 * n, dh // 4))
    h1, p2q = _tc_combine(s1, agg1.reshape(4, npad, dh // 4), deg,
                          W_neigh1, relu=True, split_h=0,
                          wn_next=W_neigh2, split_next=2)

    d2 = W_neigh2.shape[1]
    s2 = _tc_self(h1, W_self2, b2)
    agg2 = _make_sc_agg(n, nchunk, d2 // 2, 1)(src_q4, dst3,
                                               p2q.reshape(2 * n, d2 // 2))
    h2, = _tc_combine(s2, agg2.reshape(2, npad, d2 // 2), deg, None,
                      relu=False, split_h=0, add_direct=True)

    return (h2, h0, h1)
